# Initial kernel scaffold; baseline (speedup 1.0000x reference)
import jax
import jax.numpy as jnp
from jax.experimental import pallas as pl


def kernel(z, emb_weight):
    raise NotImplementedError("write your pallas kernel here")


# speedup vs baseline: 1.0120x; 1.0120x over previous
import functools

import jax
import jax.numpy as jnp
from jax import lax
from jax.experimental import pallas as pl
from jax.experimental.pallas import tpu as pltpu
from jax.experimental.pallas import tpu_sc as plsc

ROWS = 65536
CODES = 8192
CDIM = 32
TILE_R = 1024
TILE_J = 1024
NCHUNK = CODES // TILE_J

SC_NC = 2
SC_NS = 16
SC_NW = SC_NC * SC_NS
BPW = ROWS // SC_NW


def _argmin_body(z_ref, e_ref, q_ref, idx_ref, s_scr, m_scr, a_scr):
    t = pl.program_id(1)
    zr = z_ref[...]
    et = e_ref[...]

    @pl.when(t == 0)
    def _():
        s_scr[...] = jnp.sum(zr * zr, axis=1, keepdims=True)

    zr_bf = zr.astype(jnp.bfloat16)
    c = lax.dot_general(zr_bf, et, (((1,), (1,)), ((), ())),
                        preferred_element_type=jnp.float32)
    d = (s_scr[...] - 2.0 * c) + q_ref[...]

    @pl.when(t == 0)
    def _():
        m_scr[...] = d
        a_scr[...] = jnp.zeros_like(a_scr)

    @pl.when(t > 0)
    def _():
        m = m_scr[...]
        lt = d < m
        a_scr[...] = jnp.where(lt, t, a_scr[...])
        m_scr[...] = jnp.where(lt, d, m)

    @pl.when(t == NCHUNK - 1)
    def _():
        m = m_scr[...]
        dmin = jnp.min(m, axis=1, keepdims=True)
        lane = lax.broadcasted_iota(jnp.int32, m.shape, 1)
        jfull = a_scr[...] * TILE_J + lane
        cand = jnp.where(m == dmin, jfull, jnp.int32(2**30))
        idx_ref[...] = jnp.min(cand, axis=1, keepdims=True)


def _argmin_call(z_, emb_weight, q, interpret=False):
    return pl.pallas_call(
        _argmin_body,
        grid=(ROWS // TILE_R, NCHUNK),
        in_specs=[
            pl.BlockSpec((TILE_R, CDIM), lambda n, t: (n, 0)),
            pl.BlockSpec((TILE_J, CDIM), lambda n, t: (t, 0)),
            pl.BlockSpec((1, TILE_J), lambda n, t: (0, t)),
        ],
        out_specs=pl.BlockSpec((TILE_R, 1), lambda n, t: (n, 0)),
        out_shape=jax.ShapeDtypeStruct((ROWS, 1), jnp.int32),
        scratch_shapes=[
            pltpu.VMEM((TILE_R, 1), jnp.float32),
            pltpu.VMEM((TILE_R, TILE_J), jnp.float32),
            pltpu.VMEM((TILE_R, TILE_J), jnp.int32),
        ],
        interpret=interpret,
    )(z_, emb_weight, q)


@functools.cache
def _sc_gather_kernel():
    @functools.partial(
        pl.kernel,
        mesh=plsc.VectorSubcoreMesh(core_axis_name="c", subcore_axis_name="s"),
        out_type=jax.ShapeDtypeStruct((ROWS, CDIM), jnp.float32),
        scratch_types=[
            pltpu.VMEM((BPW,), jnp.int32),
            pltpu.VMEM((BPW, CDIM), jnp.float32),
            pltpu.SemaphoreType.DMA,
        ],
        compiler_params=pltpu.CompilerParams(use_tc_tiling_on_sc=False),
    )
    def _sc_gather(table_hbm, idx_hbm, out_hbm, idx_v, rows_v, sem):
        wid = lax.axis_index("s") * SC_NC + lax.axis_index("c")
        base = wid * BPW
        pltpu.sync_copy(idx_hbm.at[pl.ds(base, BPW)], idx_v)
        pltpu.async_copy(table_hbm.at[idx_v], rows_v, sem).wait()
        pltpu.sync_copy(rows_v, out_hbm.at[pl.ds(base, BPW)])

    return _sc_gather


def kernel(z, emb_weight):
    n, c, h, w = z.shape
    z_ = jnp.transpose(z, (0, 2, 3, 1)).reshape(-1, c)
    q = jnp.sum(emb_weight.T ** 2, axis=0, keepdims=True)
    idx = _argmin_call(z_, emb_weight, q).reshape(-1)
    gathered = _sc_gather_kernel()(emb_weight, idx)
    quantized = jnp.transpose(gathered.reshape(n, h, w, c), (0, 3, 1, 2))
    encoding_indices = idx.reshape(n, h, w)
    return quantized, quantized, encoding_indices
